# baseline (speedup 1.0000x reference)
import functools

import jax
import jax.numpy as jnp
from jax import lax
from jax.experimental import pallas as pl
from jax.experimental.pallas import tpu as pltpu
from jax.experimental.pallas import tpu_sc as plsc

B = 16384
T = 20
N = B * T
D_OUT = 256
TBL_ROWS = 512
NC = 2
NS = 16
NW = NC * NS
ROWS_PER_W = N // NW
CHUNK = 128
NCHUNK = ROWS_PER_W // CHUNK
NBUF = 2
NGROUP = NCHUNK // NBUF


def _prep_body(cs, cp, co, es, ep, eo, w, bias, slot, piece, orient,
               table_lo_out, table_hi_out, idx_out):
    dn = (((1,), (1,)), ((), ()))
    f32 = jnp.float32
    csp = lax.dot_general(cs[...], w[...], dn, preferred_element_type=f32)
    cpp = lax.dot_general(cp[...], w[...], dn, preferred_element_type=f32)
    cop = lax.dot_general(co[...], w[...], dn, preferred_element_type=f32)
    esp = lax.dot_general(es[...], w[...], dn, preferred_element_type=f32)
    epp = lax.dot_general(ep[...], w[...], dn, preferred_element_type=f32)
    eop = lax.dot_general(eo[...], w[...], dn, preferred_element_type=f32)

    dn2 = (((1,), (0,)), ((), ()))

    def combine(nrows, sp, pp, op, pdiv, pmod, omod):
        r_s = lax.broadcasted_iota(jnp.int32, (nrows, sp.shape[0]), 0)
        c_s = lax.broadcasted_iota(jnp.int32, (nrows, sp.shape[0]), 1)
        s_oh = (r_s // (pmod * omod) == c_s).astype(f32)
        r_p = lax.broadcasted_iota(jnp.int32, (nrows, pp.shape[0]), 0)
        c_p = lax.broadcasted_iota(jnp.int32, (nrows, pp.shape[0]), 1)
        p_oh = ((r_p // omod) % pmod == c_p).astype(f32)
        r_o = lax.broadcasted_iota(jnp.int32, (nrows, op.shape[0]), 0)
        c_o = lax.broadcasted_iota(jnp.int32, (nrows, op.shape[0]), 1)
        o_oh = (r_o % omod == c_o).astype(f32)
        return (lax.dot_general(s_oh, sp, dn2, preferred_element_type=f32)
                + lax.dot_general(p_oh, pp, dn2, preferred_element_type=f32)
                + lax.dot_general(o_oh, op, dn2, preferred_element_type=f32))

    b_row = bias[...]
    corner = combine(192, csp, cpp, cop, 8, 8, 3) + b_row
    edge = combine(288, esp, epp, eop, 12, 12, 2) + b_row
    table_lo_out[0:192, :] = corner[:, :128]
    table_lo_out[192:480, :] = edge[:, :128]
    table_lo_out[480:512, :] = jnp.zeros((32, 128), f32)
    table_hi_out[0:192, :] = corner[:, 128:]
    table_hi_out[192:480, :] = edge[:, 128:]
    table_hi_out[480:512, :] = jnp.zeros((32, 128), f32)

    row = lax.broadcasted_iota(jnp.int32, (N // 256, 256), 0)
    col = lax.broadcasted_iota(jnp.int32, (N // 256, 256), 1)
    t = (row * 256 + col) % T
    s_id = slot[...]
    p_id = piece[...]
    o_id = orient[...]
    is_c = t < 8
    corner_idx = s_id * 24 + p_id * 3 + o_id
    edge_idx = 192 + s_id * 24 + p_id * 2 + o_id
    idx_out[...] = jnp.where(is_c, corner_idx, edge_idx)


def _gather_body(tlo_hbm, thi_hbm, idx_hbm, olo_hbm, ohi_hbm,
                 idx_v, lo_v, hi_v, g0, g1, o0, o1):
    gsem = [g0, g1]
    osem = [o0, o1]
    wid = lax.axis_index("s") * NC + lax.axis_index("c")
    base = wid * ROWS_PER_W
    pltpu.sync_copy(idx_hbm.at[wid], idx_v)

    def g_start(j, b):
        pltpu.async_copy(tlo_hbm.at[idx_v.at[j]], lo_v.at[b], gsem[b])
        pltpu.async_copy(thi_hbm.at[idx_v.at[j]], hi_v.at[b], gsem[b])

    def g_wait(j, b):
        pltpu.make_async_copy(
            tlo_hbm.at[idx_v.at[j]], lo_v.at[b], gsem[b]).wait()
        pltpu.make_async_copy(
            thi_hbm.at[idx_v.at[j]], hi_v.at[b], gsem[b]).wait()

    def o_start(j, b):
        dst = pl.ds(base + j * CHUNK, CHUNK)
        pltpu.async_copy(lo_v.at[b], olo_hbm.at[dst], osem[b])
        pltpu.async_copy(hi_v.at[b], ohi_hbm.at[dst], osem[b])

    def o_wait(j, b):
        dst = pl.ds(base + j * CHUNK, CHUNK)
        pltpu.make_async_copy(lo_v.at[b], olo_hbm.at[dst], osem[b]).wait()
        pltpu.make_async_copy(hi_v.at[b], ohi_hbm.at[dst], osem[b]).wait()

    for b in range(NBUF):
        g_start(b, b)

    def group(g, carry):
        for b in range(NBUF):
            j = g * NBUF + b
            g_wait(j, b)
            o_start(j, b)

            @pl.when(g < NGROUP - 1)
            def _refill():
                o_wait(j, b)
                g_start(j + NBUF, b)

        return carry

    lax.fori_loop(0, NGROUP, group, 0)
    for b in range(NBUF):
        o_wait((NGROUP - 1) * NBUF + b, b)


NB_B = 128


def _merge_body(lo_ref, hi_ref, out_ref):
    out_ref[...] = jnp.concatenate(
        [lo_ref[...], hi_ref[...]], axis=-1).reshape(1, NB_B, D_OUT)


def kernel(slot_ids, piece_ids, orientations, corner_slot_emb,
           corner_piece_emb, corner_orient_emb, edge_slot_emb,
           edge_piece_emb, edge_orient_emb, proj_W, proj_b):
    f32 = jnp.float32

    def pad_cols(a, lo):
        return jnp.pad(a, ((0, 0), (lo, 128 - lo - a.shape[1])))

    cs = pad_cols(corner_slot_emb, 0)
    cp = pad_cols(corner_piece_emb, 42)
    co = pad_cols(corner_orient_emb, 84)
    es = pad_cols(edge_slot_emb, 0)
    ep = pad_cols(edge_piece_emb, 42)
    eo = pad_cols(edge_orient_emb, 84)

    slot_r = slot_ids.reshape(N // 256, 256)
    piece_r = piece_ids.reshape(N // 256, 256)
    orient_r = orientations.reshape(N // 256, 256)

    table_lo, table_hi, idx = pl.pallas_call(
        _prep_body,
        out_shape=[
            jax.ShapeDtypeStruct((TBL_ROWS, 128), f32),
            jax.ShapeDtypeStruct((TBL_ROWS, 128), f32),
            jax.ShapeDtypeStruct((N // 256, 256), jnp.int32),
        ],
    )(cs, cp, co, es, ep, eo, proj_W, proj_b.reshape(1, D_OUT),
      slot_r, piece_r, orient_r)

    idx_3d = idx.reshape(B, T).transpose(1, 0).reshape(NW, NCHUNK, CHUNK)

    mesh = plsc.VectorSubcoreMesh(core_axis_name="c", subcore_axis_name="s")
    gather = functools.partial(
        pl.kernel,
        mesh=mesh,
        out_type=[
            jax.ShapeDtypeStruct((N, 128), f32),
            jax.ShapeDtypeStruct((N, 128), f32),
        ],
        scratch_types=[
            pltpu.VMEM((NCHUNK, CHUNK), jnp.int32),
            pltpu.VMEM((NBUF, CHUNK, 128), f32),
            pltpu.VMEM((NBUF, CHUNK, 128), f32),
            pltpu.SemaphoreType.DMA,
            pltpu.SemaphoreType.DMA,
            pltpu.SemaphoreType.DMA,
            pltpu.SemaphoreType.DMA,
        ],
    )(_gather_body)

    out_lo, out_hi = gather(table_lo, table_hi, idx_3d)

    out_t = pl.pallas_call(
        _merge_body,
        grid=(T, B // NB_B),
        in_specs=[
            pl.BlockSpec((NB_B, 128), lambda t, i: (t * (B // NB_B) + i, 0)),
            pl.BlockSpec((NB_B, 128), lambda t, i: (t * (B // NB_B) + i, 0)),
        ],
        out_specs=pl.BlockSpec((1, NB_B, D_OUT), lambda t, i: (t, i, 0)),
        out_shape=jax.ShapeDtypeStruct((T, B, D_OUT), f32),
    )(out_lo, out_hi)
    return out_t.transpose(1, 0, 2)


# speedup vs baseline: 2.4616x; 2.4616x over previous
import functools

import jax
import jax.numpy as jnp
from jax import lax
from jax.experimental import pallas as pl
from jax.experimental.pallas import tpu as pltpu
from jax.experimental.pallas import tpu_sc as plsc

B = 16384
T = 20
N = B * T
D_OUT = 256
TBL_ROWS = 512
NC = 2
NS = 16
NW = NC * NS
ROWS_PER_W = N // NW
CHUNK = 128
NCHUNK = ROWS_PER_W // CHUNK
NBUF = 2
NGROUP = NCHUNK // NBUF


def _prep_body(cs, cp, co, es, ep, eo, w, bias, slot, piece, orient,
               table_lo_out, table_hi_out, idx_out):
    dn = (((1,), (1,)), ((), ()))
    f32 = jnp.float32
    csp = lax.dot_general(cs[...], w[...], dn, preferred_element_type=f32)
    cpp = lax.dot_general(cp[...], w[...], dn, preferred_element_type=f32)
    cop = lax.dot_general(co[...], w[...], dn, preferred_element_type=f32)
    esp = lax.dot_general(es[...], w[...], dn, preferred_element_type=f32)
    epp = lax.dot_general(ep[...], w[...], dn, preferred_element_type=f32)
    eop = lax.dot_general(eo[...], w[...], dn, preferred_element_type=f32)

    dn2 = (((1,), (0,)), ((), ()))

    def combine(nrows, sp, pp, op, pdiv, pmod, omod):
        r_s = lax.broadcasted_iota(jnp.int32, (nrows, sp.shape[0]), 0)
        c_s = lax.broadcasted_iota(jnp.int32, (nrows, sp.shape[0]), 1)
        s_oh = (r_s // (pmod * omod) == c_s).astype(f32)
        r_p = lax.broadcasted_iota(jnp.int32, (nrows, pp.shape[0]), 0)
        c_p = lax.broadcasted_iota(jnp.int32, (nrows, pp.shape[0]), 1)
        p_oh = ((r_p // omod) % pmod == c_p).astype(f32)
        r_o = lax.broadcasted_iota(jnp.int32, (nrows, op.shape[0]), 0)
        c_o = lax.broadcasted_iota(jnp.int32, (nrows, op.shape[0]), 1)
        o_oh = (r_o % omod == c_o).astype(f32)
        return (lax.dot_general(s_oh, sp, dn2, preferred_element_type=f32)
                + lax.dot_general(p_oh, pp, dn2, preferred_element_type=f32)
                + lax.dot_general(o_oh, op, dn2, preferred_element_type=f32))

    b_row = bias[...]
    corner = combine(192, csp, cpp, cop, 8, 8, 3) + b_row
    edge = combine(288, esp, epp, eop, 12, 12, 2) + b_row
    table_lo_out[0:192, :] = corner[:, :128]
    table_lo_out[192:480, :] = edge[:, :128]
    table_lo_out[480:512, :] = jnp.zeros((32, 128), f32)
    table_hi_out[0:192, :] = corner[:, 128:]
    table_hi_out[192:480, :] = edge[:, 128:]
    table_hi_out[480:512, :] = jnp.zeros((32, 128), f32)

    row = lax.broadcasted_iota(jnp.int32, (N // 256, 256), 0)
    col = lax.broadcasted_iota(jnp.int32, (N // 256, 256), 1)
    t = (row * 256 + col) % T
    s_id = slot[...]
    p_id = piece[...]
    o_id = orient[...]
    is_c = t < 8
    corner_idx = s_id * 24 + p_id * 3 + o_id
    edge_idx = 192 + s_id * 24 + p_id * 2 + o_id
    idx_out[...] = jnp.where(is_c, corner_idx, edge_idx)


def _gather_body(tlo_hbm, thi_hbm, idx_hbm, olo_hbm, ohi_hbm,
                 idx_v, lo_v, hi_v, g0, g1, o0, o1):
    gsem = [g0, g1]
    osem = [o0, o1]
    wid = lax.axis_index("s") * NC + lax.axis_index("c")
    base = wid * ROWS_PER_W
    pltpu.sync_copy(idx_hbm.at[wid], idx_v)

    def g_start(j, b):
        pltpu.async_copy(tlo_hbm.at[idx_v.at[j]], lo_v.at[b], gsem[b])
        pltpu.async_copy(thi_hbm.at[idx_v.at[j]], hi_v.at[b], gsem[b])

    def g_wait(j, b):
        pltpu.make_async_copy(
            tlo_hbm.at[idx_v.at[j]], lo_v.at[b], gsem[b]).wait()
        pltpu.make_async_copy(
            thi_hbm.at[idx_v.at[j]], hi_v.at[b], gsem[b]).wait()

    def o_start(j, b):
        dst = pl.ds(base + j * CHUNK, CHUNK)
        pltpu.async_copy(lo_v.at[b], olo_hbm.at[dst], osem[b])
        pltpu.async_copy(hi_v.at[b], ohi_hbm.at[dst], osem[b])

    def o_wait(j, b):
        dst = pl.ds(base + j * CHUNK, CHUNK)
        pltpu.make_async_copy(lo_v.at[b], olo_hbm.at[dst], osem[b]).wait()
        pltpu.make_async_copy(hi_v.at[b], ohi_hbm.at[dst], osem[b]).wait()

    for b in range(NBUF):
        g_start(b, b)

    def group(g, carry):
        for b in range(NBUF):
            j = g * NBUF + b
            g_wait(j, b)
            o_start(j, b)

            @pl.when(g < NGROUP - 1)
            def _refill():
                o_wait(j, b)
                g_start(j + NBUF, b)

        return carry

    lax.fori_loop(0, NGROUP, group, 0)
    for b in range(NBUF):
        o_wait((NGROUP - 1) * NBUF + b, b)


NB_B = 2048


def _merge_body(lo_ref, hi_ref, out_ref):
    out_ref[...] = jnp.concatenate(
        [lo_ref[...], hi_ref[...]], axis=-1).reshape(1, NB_B, D_OUT)


def kernel(slot_ids, piece_ids, orientations, corner_slot_emb,
           corner_piece_emb, corner_orient_emb, edge_slot_emb,
           edge_piece_emb, edge_orient_emb, proj_W, proj_b):
    f32 = jnp.float32

    def pad_cols(a, lo):
        return jnp.pad(a, ((0, 0), (lo, 128 - lo - a.shape[1])))

    cs = pad_cols(corner_slot_emb, 0)
    cp = pad_cols(corner_piece_emb, 42)
    co = pad_cols(corner_orient_emb, 84)
    es = pad_cols(edge_slot_emb, 0)
    ep = pad_cols(edge_piece_emb, 42)
    eo = pad_cols(edge_orient_emb, 84)

    slot_r = slot_ids.reshape(N // 256, 256)
    piece_r = piece_ids.reshape(N // 256, 256)
    orient_r = orientations.reshape(N // 256, 256)

    table_lo, table_hi, idx = pl.pallas_call(
        _prep_body,
        out_shape=[
            jax.ShapeDtypeStruct((TBL_ROWS, 128), f32),
            jax.ShapeDtypeStruct((TBL_ROWS, 128), f32),
            jax.ShapeDtypeStruct((N // 256, 256), jnp.int32),
        ],
    )(cs, cp, co, es, ep, eo, proj_W, proj_b.reshape(1, D_OUT),
      slot_r, piece_r, orient_r)

    idx_3d = idx.reshape(B, T).transpose(1, 0).reshape(NW, NCHUNK, CHUNK)

    mesh = plsc.VectorSubcoreMesh(core_axis_name="c", subcore_axis_name="s")
    gather = functools.partial(
        pl.kernel,
        mesh=mesh,
        out_type=[
            jax.ShapeDtypeStruct((N, 128), f32),
            jax.ShapeDtypeStruct((N, 128), f32),
        ],
        scratch_types=[
            pltpu.VMEM((NCHUNK, CHUNK), jnp.int32),
            pltpu.VMEM((NBUF, CHUNK, 128), f32),
            pltpu.VMEM((NBUF, CHUNK, 128), f32),
            pltpu.SemaphoreType.DMA,
            pltpu.SemaphoreType.DMA,
            pltpu.SemaphoreType.DMA,
            pltpu.SemaphoreType.DMA,
        ],
    )(_gather_body)

    out_lo, out_hi = gather(table_lo, table_hi, idx_3d)

    out_t = pl.pallas_call(
        _merge_body,
        grid=(T, B // NB_B),
        in_specs=[
            pl.BlockSpec((NB_B, 128), lambda t, i: (t * (B // NB_B) + i, 0)),
            pl.BlockSpec((NB_B, 128), lambda t, i: (t * (B // NB_B) + i, 0)),
        ],
        out_specs=pl.BlockSpec((1, NB_B, D_OUT), lambda t, i: (t, i, 0)),
        out_shape=jax.ShapeDtypeStruct((T, B, D_OUT), f32),
    )(out_lo, out_hi)
    return out_t.transpose(1, 0, 2)


# speedup vs baseline: 4.2103x; 1.7104x over previous
import functools

import jax
import jax.numpy as jnp
from jax import lax
from jax.experimental import pallas as pl
from jax.experimental.pallas import tpu as pltpu
from jax.experimental.pallas import tpu_sc as plsc

B = 16384
T = 20
N = B * T
D_OUT = 256
TBL_ROWS = 512
NC = 2
NS = 16
NW = NC * NS
ROWS_PER_W = N // NW
CHUNK = 128
NCHUNK = ROWS_PER_W // CHUNK
NBUF = 2
NGROUP = NCHUNK // NBUF


def _prep_body(cs, cp, co, es, ep, eo, w, bias, slot, piece, orient,
               table_lo_out, table_hi_out, idx_out):
    dn = (((1,), (1,)), ((), ()))
    f32 = jnp.float32
    csp = lax.dot_general(cs[...], w[...], dn, preferred_element_type=f32)
    cpp = lax.dot_general(cp[...], w[...], dn, preferred_element_type=f32)
    cop = lax.dot_general(co[...], w[...], dn, preferred_element_type=f32)
    esp = lax.dot_general(es[...], w[...], dn, preferred_element_type=f32)
    epp = lax.dot_general(ep[...], w[...], dn, preferred_element_type=f32)
    eop = lax.dot_general(eo[...], w[...], dn, preferred_element_type=f32)

    dn2 = (((1,), (0,)), ((), ()))

    def combine(nrows, sp, pp, op, pdiv, pmod, omod):
        r_s = lax.broadcasted_iota(jnp.int32, (nrows, sp.shape[0]), 0)
        c_s = lax.broadcasted_iota(jnp.int32, (nrows, sp.shape[0]), 1)
        s_oh = (r_s // (pmod * omod) == c_s).astype(f32)
        r_p = lax.broadcasted_iota(jnp.int32, (nrows, pp.shape[0]), 0)
        c_p = lax.broadcasted_iota(jnp.int32, (nrows, pp.shape[0]), 1)
        p_oh = ((r_p // omod) % pmod == c_p).astype(f32)
        r_o = lax.broadcasted_iota(jnp.int32, (nrows, op.shape[0]), 0)
        c_o = lax.broadcasted_iota(jnp.int32, (nrows, op.shape[0]), 1)
        o_oh = (r_o % omod == c_o).astype(f32)
        return (lax.dot_general(s_oh, sp, dn2, preferred_element_type=f32)
                + lax.dot_general(p_oh, pp, dn2, preferred_element_type=f32)
                + lax.dot_general(o_oh, op, dn2, preferred_element_type=f32))

    b_row = bias[...]
    corner = combine(192, csp, cpp, cop, 8, 8, 3) + b_row
    edge = combine(288, esp, epp, eop, 12, 12, 2) + b_row
    table_lo_out[0:192, :] = corner[:, :128]
    table_lo_out[192:480, :] = edge[:, :128]
    table_lo_out[480:512, :] = jnp.zeros((32, 128), f32)
    table_hi_out[0:192, :] = corner[:, 128:]
    table_hi_out[192:480, :] = edge[:, 128:]
    table_hi_out[480:512, :] = jnp.zeros((32, 128), f32)

    row = lax.broadcasted_iota(jnp.int32, (N // 256, 256), 0)
    col = lax.broadcasted_iota(jnp.int32, (N // 256, 256), 1)
    t = (row * 256 + col) % T
    s_id = slot[...]
    p_id = piece[...]
    o_id = orient[...]
    is_c = t < 8
    corner_idx = s_id * 24 + p_id * 3 + o_id
    edge_idx = 192 + s_id * 24 + p_id * 2 + o_id
    idx_out[...] = jnp.where(is_c, corner_idx, edge_idx)


def _gather_body(tlo_hbm, thi_hbm, idx_hbm, olo_hbm, ohi_hbm,
                 idx_v, lo_v, hi_v, tlo_sh, thi_sh, g0, g1, o0, o1):
    gsem = [g0, g1]
    osem = [o0, o1]
    wid = lax.axis_index("s") * NC + lax.axis_index("c")
    base = wid * ROWS_PER_W

    @pl.when(lax.axis_index("s") == 0)
    def _stage():
        for k in range(TBL_ROWS // CHUNK):
            sl = pl.ds(k * CHUNK, CHUNK)
            pltpu.sync_copy(tlo_hbm.at[sl], lo_v.at[0])
            pltpu.sync_copy(lo_v.at[0], tlo_sh.at[sl])
            pltpu.sync_copy(thi_hbm.at[sl], hi_v.at[0])
            pltpu.sync_copy(hi_v.at[0], thi_sh.at[sl])

    pltpu.sync_copy(idx_hbm.at[wid], idx_v)
    plsc.subcore_barrier()

    def g_start(j, b):
        pltpu.async_copy(tlo_sh.at[idx_v.at[j]], lo_v.at[b], gsem[b])
        pltpu.async_copy(thi_sh.at[idx_v.at[j]], hi_v.at[b], gsem[b])

    def g_wait(j, b):
        pltpu.make_async_copy(
            tlo_sh.at[idx_v.at[j]], lo_v.at[b], gsem[b]).wait()
        pltpu.make_async_copy(
            thi_sh.at[idx_v.at[j]], hi_v.at[b], gsem[b]).wait()

    def o_start(j, b):
        dst = pl.ds(base + j * CHUNK, CHUNK)
        pltpu.async_copy(lo_v.at[b], olo_hbm.at[dst], osem[b])
        pltpu.async_copy(hi_v.at[b], ohi_hbm.at[dst], osem[b])

    def o_wait(j, b):
        dst = pl.ds(base + j * CHUNK, CHUNK)
        pltpu.make_async_copy(lo_v.at[b], olo_hbm.at[dst], osem[b]).wait()
        pltpu.make_async_copy(hi_v.at[b], ohi_hbm.at[dst], osem[b]).wait()

    for b in range(NBUF):
        g_start(b, b)

    def group(g, carry):
        for b in range(NBUF):
            j = g * NBUF + b
            g_wait(j, b)
            o_start(j, b)

            @pl.when(g < NGROUP - 1)
            def _refill():
                o_wait(j, b)
                g_start(j + NBUF, b)

        return carry

    lax.fori_loop(0, NGROUP, group, 0)
    for b in range(NBUF):
        o_wait((NGROUP - 1) * NBUF + b, b)


NB_B = 2048


def _merge_body(lo_ref, hi_ref, out_ref):
    out_ref[...] = jnp.concatenate(
        [lo_ref[...], hi_ref[...]], axis=-1).reshape(1, NB_B, D_OUT)


def kernel(slot_ids, piece_ids, orientations, corner_slot_emb,
           corner_piece_emb, corner_orient_emb, edge_slot_emb,
           edge_piece_emb, edge_orient_emb, proj_W, proj_b):
    f32 = jnp.float32

    def pad_cols(a, lo):
        return jnp.pad(a, ((0, 0), (lo, 128 - lo - a.shape[1])))

    cs = pad_cols(corner_slot_emb, 0)
    cp = pad_cols(corner_piece_emb, 42)
    co = pad_cols(corner_orient_emb, 84)
    es = pad_cols(edge_slot_emb, 0)
    ep = pad_cols(edge_piece_emb, 42)
    eo = pad_cols(edge_orient_emb, 84)

    slot_r = slot_ids.reshape(N // 256, 256)
    piece_r = piece_ids.reshape(N // 256, 256)
    orient_r = orientations.reshape(N // 256, 256)

    table_lo, table_hi, idx = pl.pallas_call(
        _prep_body,
        out_shape=[
            jax.ShapeDtypeStruct((TBL_ROWS, 128), f32),
            jax.ShapeDtypeStruct((TBL_ROWS, 128), f32),
            jax.ShapeDtypeStruct((N // 256, 256), jnp.int32),
        ],
    )(cs, cp, co, es, ep, eo, proj_W, proj_b.reshape(1, D_OUT),
      slot_r, piece_r, orient_r)

    idx_3d = idx.reshape(B, T).transpose(1, 0).reshape(NW, NCHUNK, CHUNK)

    mesh = plsc.VectorSubcoreMesh(core_axis_name="c", subcore_axis_name="s")
    gather = functools.partial(
        pl.kernel,
        mesh=mesh,
        out_type=[
            jax.ShapeDtypeStruct((N, 128), f32),
            jax.ShapeDtypeStruct((N, 128), f32),
        ],
        scratch_types=[
            pltpu.VMEM((NCHUNK, CHUNK), jnp.int32),
            pltpu.VMEM((NBUF, CHUNK, 128), f32),
            pltpu.VMEM((NBUF, CHUNK, 128), f32),
            pltpu.VMEM_SHARED((TBL_ROWS, 128), f32),
            pltpu.VMEM_SHARED((TBL_ROWS, 128), f32),
            pltpu.SemaphoreType.DMA,
            pltpu.SemaphoreType.DMA,
            pltpu.SemaphoreType.DMA,
            pltpu.SemaphoreType.DMA,
        ],
    )(_gather_body)

    out_lo, out_hi = gather(table_lo, table_hi, idx_3d)

    out_t = pl.pallas_call(
        _merge_body,
        grid=(T, B // NB_B),
        in_specs=[
            pl.BlockSpec((NB_B, 128), lambda t, i: (t * (B // NB_B) + i, 0)),
            pl.BlockSpec((NB_B, 128), lambda t, i: (t * (B // NB_B) + i, 0)),
        ],
        out_specs=pl.BlockSpec((1, NB_B, D_OUT), lambda t, i: (t, i, 0)),
        out_shape=jax.ShapeDtypeStruct((T, B, D_OUT), f32),
    )(out_lo, out_hi)
    return out_t.transpose(1, 0, 2)


# speedup vs baseline: 9.3159x; 2.2127x over previous
import functools

import jax
import jax.numpy as jnp
from jax import lax
from jax.experimental import pallas as pl
from jax.experimental.pallas import tpu as pltpu
from jax.experimental.pallas import tpu_sc as plsc

B = 16384
T = 20
N = B * T
D_OUT = 256
TBL_ROWS = 1024
NC = 2
NS = 16
NW = NC * NS
TOK_PER_W = N // NW
OUT_PER_W = 2 * TOK_PER_W
CHUNK = 128
TOK_PER_CHUNK = CHUNK // 2
NCHUNK = OUT_PER_W // CHUNK
NBUF = 2
NGROUP = NCHUNK // NBUF


def _prep_body(cs, cp, co, es, ep, eo, w, bias, slot, piece, orient,
               table_out, idx_out):
    dn = (((1,), (1,)), ((), ()))
    f32 = jnp.float32
    csp = lax.dot_general(cs[...], w[...], dn, preferred_element_type=f32)
    cpp = lax.dot_general(cp[...], w[...], dn, preferred_element_type=f32)
    cop = lax.dot_general(co[...], w[...], dn, preferred_element_type=f32)
    esp = lax.dot_general(es[...], w[...], dn, preferred_element_type=f32)
    epp = lax.dot_general(ep[...], w[...], dn, preferred_element_type=f32)
    eop = lax.dot_general(eo[...], w[...], dn, preferred_element_type=f32)

    dn2 = (((1,), (0,)), ((), ()))

    def combine(nrows, sp, pp, op, pmod, omod):
        r_s = lax.broadcasted_iota(jnp.int32, (nrows, sp.shape[0]), 0)
        c_s = lax.broadcasted_iota(jnp.int32, (nrows, sp.shape[0]), 1)
        s_oh = (r_s // (pmod * omod) == c_s).astype(f32)
        r_p = lax.broadcasted_iota(jnp.int32, (nrows, pp.shape[0]), 0)
        c_p = lax.broadcasted_iota(jnp.int32, (nrows, pp.shape[0]), 1)
        p_oh = ((r_p // omod) % pmod == c_p).astype(f32)
        r_o = lax.broadcasted_iota(jnp.int32, (nrows, op.shape[0]), 0)
        c_o = lax.broadcasted_iota(jnp.int32, (nrows, op.shape[0]), 1)
        o_oh = (r_o % omod == c_o).astype(f32)
        return (lax.dot_general(s_oh, sp, dn2, preferred_element_type=f32)
                + lax.dot_general(p_oh, pp, dn2, preferred_element_type=f32)
                + lax.dot_general(o_oh, op, dn2, preferred_element_type=f32))

    b_row = bias[...]
    corner = combine(192, csp, cpp, cop, 8, 3) + b_row
    edge = combine(288, esp, epp, eop, 12, 2) + b_row
    zeros = jnp.zeros((32, 128), f32)
    table_out[0:192, :] = corner[:, :128]
    table_out[192:480, :] = edge[:, :128]
    table_out[480:512, :] = zeros
    table_out[512:704, :] = corner[:, 128:]
    table_out[704:992, :] = edge[:, 128:]
    table_out[992:1024, :] = zeros

    r_it = lax.broadcasted_iota(jnp.int32, (N // 64, 64), 0)
    t = r_it // 256
    s_id = slot[...]
    p_id = piece[...]
    o_id = orient[...]
    is_c = t < 8
    corner_idx = s_id * 24 + p_id * 3 + o_id
    edge_idx = 192 + s_id * 24 + p_id * 2 + o_id
    base = jnp.where(is_c, corner_idx, edge_idx).astype(f32)

    po = lax.broadcasted_iota(jnp.int32, (64, 128), 0)
    pc = lax.broadcasted_iota(jnp.int32, (64, 128), 1)
    perm = (po == 8 * (pc // 16) + (pc % 8)).astype(f32)
    c1 = lax.broadcasted_iota(jnp.int32, (1, 128), 1)
    halfoff = ((c1 // 8) % 2) * (TBL_ROWS // 2)
    idx_out[...] = lax.dot_general(
        base, perm, dn2, preferred_element_type=f32).astype(jnp.int32) + halfoff


def _gather_body(tbl_hbm, idx_hbm, out_hbm,
                 idx_v, rows_v, tbl_sh, g0, g1, o0, o1):
    gsem = [g0, g1]
    osem = [o0, o1]
    wid = lax.axis_index("s") * NC + lax.axis_index("c")
    base = wid * OUT_PER_W

    @pl.when(lax.axis_index("s") == 0)
    def _stage():
        for k in range(TBL_ROWS // CHUNK):
            sl = pl.ds(k * CHUNK, CHUNK)
            pltpu.sync_copy(tbl_hbm.at[sl], rows_v.at[0])
            pltpu.sync_copy(rows_v.at[0], tbl_sh.at[sl])

    pltpu.sync_copy(idx_hbm.at[wid], idx_v)
    plsc.subcore_barrier()

    def g_start(j, b):
        pltpu.async_copy(tbl_sh.at[idx_v.at[j]], rows_v.at[b], gsem[b])

    def g_wait(j, b):
        pltpu.make_async_copy(
            tbl_sh.at[idx_v.at[j]], rows_v.at[b], gsem[b]).wait()

    def o_start(j, b):
        dst = pl.ds(base + j * CHUNK, CHUNK)
        pltpu.async_copy(rows_v.at[b], out_hbm.at[dst], osem[b])

    def o_wait(j, b):
        dst = pl.ds(base + j * CHUNK, CHUNK)
        pltpu.make_async_copy(rows_v.at[b], out_hbm.at[dst], osem[b]).wait()

    for b in range(NBUF):
        g_start(b, b)

    def group(g, carry):
        for b in range(NBUF):
            j = g * NBUF + b
            g_wait(j, b)
            o_start(j, b)

            @pl.when(g < NGROUP - 1)
            def _refill():
                o_wait(j, b)
                g_start(j + NBUF, b)

        return carry

    lax.fori_loop(0, NGROUP, group, 0)
    for b in range(NBUF):
        o_wait((NGROUP - 1) * NBUF + b, b)


def kernel(slot_ids, piece_ids, orientations, corner_slot_emb,
           corner_piece_emb, corner_orient_emb, edge_slot_emb,
           edge_piece_emb, edge_orient_emb, proj_W, proj_b):
    f32 = jnp.float32

    def pad_cols(a, lo):
        return jnp.pad(a, ((0, 0), (lo, 128 - lo - a.shape[1])))

    cs = pad_cols(corner_slot_emb, 0)
    cp = pad_cols(corner_piece_emb, 42)
    co = pad_cols(corner_orient_emb, 84)
    es = pad_cols(edge_slot_emb, 0)
    ep = pad_cols(edge_piece_emb, 42)
    eo = pad_cols(edge_orient_emb, 84)

    slot_r = slot_ids.transpose(1, 0).reshape(N // 64, 64)
    piece_r = piece_ids.transpose(1, 0).reshape(N // 64, 64)
    orient_r = orientations.transpose(1, 0).reshape(N // 64, 64)

    table, idx = pl.pallas_call(
        _prep_body,
        out_shape=[
            jax.ShapeDtypeStruct((TBL_ROWS, 128), f32),
            jax.ShapeDtypeStruct((N // 64, 128), jnp.int32),
        ],
    )(cs, cp, co, es, ep, eo, proj_W, proj_b.reshape(1, D_OUT),
      slot_r, piece_r, orient_r)

    idx_3d = idx.reshape(NW, NCHUNK, CHUNK)

    mesh = plsc.VectorSubcoreMesh(core_axis_name="c", subcore_axis_name="s")
    gather = functools.partial(
        pl.kernel,
        mesh=mesh,
        out_type=jax.ShapeDtypeStruct((2 * N, 128), f32),
        scratch_types=[
            pltpu.VMEM((NCHUNK, CHUNK), jnp.int32),
            pltpu.VMEM((NBUF, CHUNK, 128), f32),
            pltpu.VMEM_SHARED((TBL_ROWS, 128), f32),
            pltpu.SemaphoreType.DMA,
            pltpu.SemaphoreType.DMA,
            pltpu.SemaphoreType.DMA,
            pltpu.SemaphoreType.DMA,
        ],
    )(_gather_body)

    out = gather(table, idx_3d)

    return (out.reshape(T, B // 8, 2, 8, 128)
            .transpose(1, 3, 0, 2, 4)
            .reshape(B, T, D_OUT))


# speedup vs baseline: 10.6158x; 1.1395x over previous
import functools

import jax
import jax.numpy as jnp
from jax import lax
from jax.experimental import pallas as pl
from jax.experimental.pallas import tpu as pltpu
from jax.experimental.pallas import tpu_sc as plsc

B = 16384
T = 20
N = B * T
D_OUT = 256
TBL_ROWS = 1024
NC = 2
NS = 16
NW = NC * NS
TOK_PER_W = N // NW
OUT_PER_W = 2 * TOK_PER_W
CHUNK = 128
TOK_PER_CHUNK = CHUNK // 2
NCHUNK = OUT_PER_W // CHUNK
NBUF = 2
NGROUP = NCHUNK // NBUF


def _prep_body(cs, cp, co, es, ep, eo, w, bias, slot, piece, orient,
               table_out, idx_out):
    dn = (((1,), (1,)), ((), ()))
    f32 = jnp.float32
    csp = lax.dot_general(cs[...], w[...], dn, preferred_element_type=f32)
    cpp = lax.dot_general(cp[...], w[...], dn, preferred_element_type=f32)
    cop = lax.dot_general(co[...], w[...], dn, preferred_element_type=f32)
    esp = lax.dot_general(es[...], w[...], dn, preferred_element_type=f32)
    epp = lax.dot_general(ep[...], w[...], dn, preferred_element_type=f32)
    eop = lax.dot_general(eo[...], w[...], dn, preferred_element_type=f32)

    dn2 = (((1,), (0,)), ((), ()))

    def combine(nrows, sp, pp, op, pmod, omod):
        r_s = lax.broadcasted_iota(jnp.int32, (nrows, sp.shape[0]), 0)
        c_s = lax.broadcasted_iota(jnp.int32, (nrows, sp.shape[0]), 1)
        s_oh = (r_s // (pmod * omod) == c_s).astype(f32)
        r_p = lax.broadcasted_iota(jnp.int32, (nrows, pp.shape[0]), 0)
        c_p = lax.broadcasted_iota(jnp.int32, (nrows, pp.shape[0]), 1)
        p_oh = ((r_p // omod) % pmod == c_p).astype(f32)
        r_o = lax.broadcasted_iota(jnp.int32, (nrows, op.shape[0]), 0)
        c_o = lax.broadcasted_iota(jnp.int32, (nrows, op.shape[0]), 1)
        o_oh = (r_o % omod == c_o).astype(f32)
        return (lax.dot_general(s_oh, sp, dn2, preferred_element_type=f32)
                + lax.dot_general(p_oh, pp, dn2, preferred_element_type=f32)
                + lax.dot_general(o_oh, op, dn2, preferred_element_type=f32))

    b_row = bias[...]
    corner = combine(192, csp, cpp, cop, 8, 3) + b_row
    edge = combine(288, esp, epp, eop, 12, 2) + b_row
    zeros = jnp.zeros((32, 128), f32)
    table_out[0:192, :] = corner[:, :128]
    table_out[192:480, :] = edge[:, :128]
    table_out[480:512, :] = zeros
    table_out[512:704, :] = corner[:, 128:]
    table_out[704:992, :] = edge[:, 128:]
    table_out[992:1024, :] = zeros

    r_it = lax.broadcasted_iota(jnp.int32, (N // 64, 64), 0)
    t = r_it // 256
    s_id = slot[...]
    p_id = piece[...]
    o_id = orient[...]
    is_c = t < 8
    corner_idx = s_id * 24 + p_id * 3 + o_id
    edge_idx = 192 + s_id * 24 + p_id * 2 + o_id
    base = jnp.where(is_c, corner_idx, edge_idx).astype(f32)

    po = lax.broadcasted_iota(jnp.int32, (64, 128), 0)
    pc = lax.broadcasted_iota(jnp.int32, (64, 128), 1)
    perm = (po == 8 * (pc // 16) + (pc % 8)).astype(f32)
    c1 = lax.broadcasted_iota(jnp.int32, (1, 128), 1)
    halfoff = ((c1 // 8) % 2) * (TBL_ROWS // 2)
    expanded = lax.dot_general(base, perm, dn2, preferred_element_type=f32,
                               precision=lax.Precision.HIGHEST)
    idx_out[...] = (expanded + 0.5).astype(jnp.int32) + halfoff


def _gather_body(tbl_hbm, idx_hbm, out_hbm,
                 idx_v, rows_v, tbl_sh, g0, g1, o0, o1):
    gsem = [g0, g1]
    osem = [o0, o1]
    wid = lax.axis_index("s") * NC + lax.axis_index("c")
    base = wid * OUT_PER_W

    @pl.when(lax.axis_index("s") == 0)
    def _stage():
        for k in range(TBL_ROWS // CHUNK):
            sl = pl.ds(k * CHUNK, CHUNK)
            pltpu.sync_copy(tbl_hbm.at[sl], rows_v.at[0])
            pltpu.sync_copy(rows_v.at[0], tbl_sh.at[sl])

    pltpu.sync_copy(idx_hbm.at[wid], idx_v)
    plsc.subcore_barrier()

    def g_start(j, b):
        pltpu.async_copy(tbl_sh.at[idx_v.at[j]], rows_v.at[b], gsem[b])

    def g_wait(j, b):
        pltpu.make_async_copy(
            tbl_sh.at[idx_v.at[j]], rows_v.at[b], gsem[b]).wait()

    def o_start(j, b):
        dst = pl.ds(base + j * CHUNK, CHUNK)
        pltpu.async_copy(rows_v.at[b], out_hbm.at[dst], osem[b])

    def o_wait(j, b):
        dst = pl.ds(base + j * CHUNK, CHUNK)
        pltpu.make_async_copy(rows_v.at[b], out_hbm.at[dst], osem[b]).wait()

    for b in range(NBUF):
        g_start(b, b)

    def group(g, carry):
        for b in range(NBUF):
            j = g * NBUF + b
            g_wait(j, b)
            o_start(j, b)

            @pl.when(g < NGROUP - 1)
            def _refill():
                o_wait(j, b)
                g_start(j + NBUF, b)

        return carry

    lax.fori_loop(0, NGROUP, group, 0)
    for b in range(NBUF):
        o_wait((NGROUP - 1) * NBUF + b, b)


def kernel(slot_ids, piece_ids, orientations, corner_slot_emb,
           corner_piece_emb, corner_orient_emb, edge_slot_emb,
           edge_piece_emb, edge_orient_emb, proj_W, proj_b):
    f32 = jnp.float32

    def pad_cols(a, lo):
        return jnp.pad(a, ((0, 0), (lo, 128 - lo - a.shape[1])))

    cs = pad_cols(corner_slot_emb, 0)
    cp = pad_cols(corner_piece_emb, 42)
    co = pad_cols(corner_orient_emb, 84)
    es = pad_cols(edge_slot_emb, 0)
    ep = pad_cols(edge_piece_emb, 42)
    eo = pad_cols(edge_orient_emb, 84)

    slot_r = slot_ids.transpose(1, 0).reshape(N // 64, 64)
    piece_r = piece_ids.transpose(1, 0).reshape(N // 64, 64)
    orient_r = orientations.transpose(1, 0).reshape(N // 64, 64)

    table, idx = pl.pallas_call(
        _prep_body,
        out_shape=[
            jax.ShapeDtypeStruct((TBL_ROWS, 128), f32),
            jax.ShapeDtypeStruct((N // 64, 128), jnp.int32),
        ],
    )(cs, cp, co, es, ep, eo, proj_W, proj_b.reshape(1, D_OUT),
      slot_r, piece_r, orient_r)

    idx_3d = idx.reshape(NW, NCHUNK, CHUNK)

    mesh = plsc.VectorSubcoreMesh(core_axis_name="c", subcore_axis_name="s")
    gather = functools.partial(
        pl.kernel,
        mesh=mesh,
        out_type=jax.ShapeDtypeStruct((2 * N, 128), f32),
        scratch_types=[
            pltpu.VMEM((NCHUNK, CHUNK), jnp.int32),
            pltpu.VMEM((NBUF, CHUNK, 128), f32),
            pltpu.VMEM_SHARED((TBL_ROWS, 128), f32),
            pltpu.SemaphoreType.DMA,
            pltpu.SemaphoreType.DMA,
            pltpu.SemaphoreType.DMA,
            pltpu.SemaphoreType.DMA,
        ],
    )(_gather_body)

    out = gather(table, idx_3d)

    return (out.reshape(T, B // 8, 2, 8, 128)
            .transpose(1, 3, 0, 2, 4)
            .reshape(B, T, D_OUT))


# speedup vs baseline: 10.7927x; 1.0167x over previous
import functools

import jax
import jax.numpy as jnp
from jax import lax
from jax.experimental import pallas as pl
from jax.experimental.pallas import tpu as pltpu
from jax.experimental.pallas import tpu_sc as plsc

B = 16384
T = 20
N = B * T
D_OUT = 256
TBL_ROWS = 1024
NC = 2
NS = 16
NW = NC * NS
TOK_PER_W = N // NW
OUT_PER_W = 2 * TOK_PER_W
CHUNK = 128
TOK_PER_CHUNK = CHUNK // 2
NCHUNK = OUT_PER_W // CHUNK
NBUF = 4
NGROUP = NCHUNK // NBUF


def _prep_body(cs, cp, co, es, ep, eo, w, bias, slot, piece, orient,
               table_out, idx_out):
    dn = (((1,), (1,)), ((), ()))
    f32 = jnp.float32
    csp = lax.dot_general(cs[...], w[...], dn, preferred_element_type=f32)
    cpp = lax.dot_general(cp[...], w[...], dn, preferred_element_type=f32)
    cop = lax.dot_general(co[...], w[...], dn, preferred_element_type=f32)
    esp = lax.dot_general(es[...], w[...], dn, preferred_element_type=f32)
    epp = lax.dot_general(ep[...], w[...], dn, preferred_element_type=f32)
    eop = lax.dot_general(eo[...], w[...], dn, preferred_element_type=f32)

    dn2 = (((1,), (0,)), ((), ()))

    def combine(nrows, sp, pp, op, pmod, omod):
        r_s = lax.broadcasted_iota(jnp.int32, (nrows, sp.shape[0]), 0)
        c_s = lax.broadcasted_iota(jnp.int32, (nrows, sp.shape[0]), 1)
        s_oh = (r_s // (pmod * omod) == c_s).astype(f32)
        r_p = lax.broadcasted_iota(jnp.int32, (nrows, pp.shape[0]), 0)
        c_p = lax.broadcasted_iota(jnp.int32, (nrows, pp.shape[0]), 1)
        p_oh = ((r_p // omod) % pmod == c_p).astype(f32)
        r_o = lax.broadcasted_iota(jnp.int32, (nrows, op.shape[0]), 0)
        c_o = lax.broadcasted_iota(jnp.int32, (nrows, op.shape[0]), 1)
        o_oh = (r_o % omod == c_o).astype(f32)
        return (lax.dot_general(s_oh, sp, dn2, preferred_element_type=f32)
                + lax.dot_general(p_oh, pp, dn2, preferred_element_type=f32)
                + lax.dot_general(o_oh, op, dn2, preferred_element_type=f32))

    b_row = bias[...]
    corner = combine(192, csp, cpp, cop, 8, 3) + b_row
    edge = combine(288, esp, epp, eop, 12, 2) + b_row
    zeros = jnp.zeros((32, 128), f32)
    table_out[0:192, :] = corner[:, :128]
    table_out[192:480, :] = edge[:, :128]
    table_out[480:512, :] = zeros
    table_out[512:704, :] = corner[:, 128:]
    table_out[704:992, :] = edge[:, 128:]
    table_out[992:1024, :] = zeros

    r_it = lax.broadcasted_iota(jnp.int32, (N // 64, 64), 0)
    t = r_it // 256
    s_id = slot[...]
    p_id = piece[...]
    o_id = orient[...]
    is_c = t < 8
    corner_idx = s_id * 24 + p_id * 3 + o_id
    edge_idx = 192 + s_id * 24 + p_id * 2 + o_id
    base = jnp.where(is_c, corner_idx, edge_idx).astype(f32)

    po = lax.broadcasted_iota(jnp.int32, (64, 128), 0)
    pc = lax.broadcasted_iota(jnp.int32, (64, 128), 1)
    perm = (po == 8 * (pc // 16) + (pc % 8)).astype(f32)
    c1 = lax.broadcasted_iota(jnp.int32, (1, 128), 1)
    halfoff = ((c1 // 8) % 2) * (TBL_ROWS // 2)
    expanded = lax.dot_general(base, perm, dn2, preferred_element_type=f32,
                               precision=lax.Precision.HIGHEST)
    idx_out[...] = (expanded + 0.5).astype(jnp.int32) + halfoff


def _gather_body(tbl_hbm, idx_hbm, out_hbm,
                 idx_v, rows_v, tbl_sh, g0, g1, g2, g3, o0, o1, o2, o3):
    gsem = [g0, g1, g2, g3]
    osem = [o0, o1, o2, o3]
    wid = lax.axis_index("s") * NC + lax.axis_index("c")
    base = wid * OUT_PER_W

    @pl.when(lax.axis_index("s") == 0)
    def _stage():
        for k in range(TBL_ROWS // CHUNK):
            sl = pl.ds(k * CHUNK, CHUNK)
            pltpu.sync_copy(tbl_hbm.at[sl], rows_v.at[0])
            pltpu.sync_copy(rows_v.at[0], tbl_sh.at[sl])

    pltpu.sync_copy(idx_hbm.at[wid], idx_v)
    plsc.subcore_barrier()

    def g_start(j, b):
        pltpu.async_copy(tbl_sh.at[idx_v.at[j]], rows_v.at[b], gsem[b])

    def g_wait(j, b):
        pltpu.make_async_copy(
            tbl_sh.at[idx_v.at[j]], rows_v.at[b], gsem[b]).wait()

    def o_start(j, b):
        dst = pl.ds(base + j * CHUNK, CHUNK)
        pltpu.async_copy(rows_v.at[b], out_hbm.at[dst], osem[b])

    def o_wait(j, b):
        dst = pl.ds(base + j * CHUNK, CHUNK)
        pltpu.make_async_copy(rows_v.at[b], out_hbm.at[dst], osem[b]).wait()

    for b in range(NBUF):
        g_start(b, b)

    def group(g, carry):
        for b in range(NBUF):
            j = g * NBUF + b
            g_wait(j, b)
            o_start(j, b)

            @pl.when(g < NGROUP - 1)
            def _refill():
                o_wait(j, b)
                g_start(j + NBUF, b)

        return carry

    lax.fori_loop(0, NGROUP, group, 0)
    for b in range(NBUF):
        o_wait((NGROUP - 1) * NBUF + b, b)


def kernel(slot_ids, piece_ids, orientations, corner_slot_emb,
           corner_piece_emb, corner_orient_emb, edge_slot_emb,
           edge_piece_emb, edge_orient_emb, proj_W, proj_b):
    f32 = jnp.float32

    def pad_cols(a, lo):
        return jnp.pad(a, ((0, 0), (lo, 128 - lo - a.shape[1])))

    cs = pad_cols(corner_slot_emb, 0)
    cp = pad_cols(corner_piece_emb, 42)
    co = pad_cols(corner_orient_emb, 84)
    es = pad_cols(edge_slot_emb, 0)
    ep = pad_cols(edge_piece_emb, 42)
    eo = pad_cols(edge_orient_emb, 84)

    slot_r = slot_ids.transpose(1, 0).reshape(N // 64, 64)
    piece_r = piece_ids.transpose(1, 0).reshape(N // 64, 64)
    orient_r = orientations.transpose(1, 0).reshape(N // 64, 64)

    table, idx = pl.pallas_call(
        _prep_body,
        out_shape=[
            jax.ShapeDtypeStruct((TBL_ROWS, 128), f32),
            jax.ShapeDtypeStruct((N // 64, 128), jnp.int32),
        ],
    )(cs, cp, co, es, ep, eo, proj_W, proj_b.reshape(1, D_OUT),
      slot_r, piece_r, orient_r)

    idx_3d = idx.reshape(NW, NCHUNK, CHUNK)

    mesh = plsc.VectorSubcoreMesh(core_axis_name="c", subcore_axis_name="s")
    gather = functools.partial(
        pl.kernel,
        mesh=mesh,
        out_type=jax.ShapeDtypeStruct((2 * N, 128), f32),
        scratch_types=[
            pltpu.VMEM((NCHUNK, CHUNK), jnp.int32),
            pltpu.VMEM((NBUF, CHUNK, 128), f32),
            pltpu.VMEM_SHARED((TBL_ROWS, 128), f32),
            pltpu.SemaphoreType.DMA,
            pltpu.SemaphoreType.DMA,
            pltpu.SemaphoreType.DMA,
            pltpu.SemaphoreType.DMA,
            pltpu.SemaphoreType.DMA,
            pltpu.SemaphoreType.DMA,
            pltpu.SemaphoreType.DMA,
            pltpu.SemaphoreType.DMA,
        ],
    )(_gather_body)

    out = gather(table, idx_3d)

    return (out.reshape(T, B // 8, 2, 8, 128)
            .transpose(1, 3, 0, 2, 4)
            .reshape(B, T, D_OUT))


# speedup vs baseline: 11.6362x; 1.0782x over previous
import functools

import jax
import jax.numpy as jnp
from jax import lax
from jax.experimental import pallas as pl
from jax.experimental.pallas import tpu as pltpu
from jax.experimental.pallas import tpu_sc as plsc

B = 16384
T = 20
N = B * T
D_OUT = 256
TBL_ROWS = 1024
NC = 2
NS = 16
NW = NC * NS
TOK_PER_W = N // NW
OUT_PER_W = 2 * TOK_PER_W
CHUNK = 128
TOK_PER_CHUNK = CHUNK // 2
NCHUNK = OUT_PER_W // CHUNK
NBUF = 4
NGROUP = NCHUNK // NBUF


def _prep_body(cs, cp, co, es, ep, eo, w, bias, slot, piece, orient,
               table_out, idx_out):
    dn = (((1,), (1,)), ((), ()))
    f32 = jnp.float32
    csp = lax.dot_general(cs[...], w[...], dn, preferred_element_type=f32)
    cpp = lax.dot_general(cp[...], w[...], dn, preferred_element_type=f32)
    cop = lax.dot_general(co[...], w[...], dn, preferred_element_type=f32)
    esp = lax.dot_general(es[...], w[...], dn, preferred_element_type=f32)
    epp = lax.dot_general(ep[...], w[...], dn, preferred_element_type=f32)
    eop = lax.dot_general(eo[...], w[...], dn, preferred_element_type=f32)

    dn2 = (((1,), (0,)), ((), ()))

    def combine(nrows, sp, pp, op, pmod, omod):
        r_s = lax.broadcasted_iota(jnp.int32, (nrows, sp.shape[0]), 0)
        c_s = lax.broadcasted_iota(jnp.int32, (nrows, sp.shape[0]), 1)
        s_oh = (r_s // (pmod * omod) == c_s).astype(f32)
        r_p = lax.broadcasted_iota(jnp.int32, (nrows, pp.shape[0]), 0)
        c_p = lax.broadcasted_iota(jnp.int32, (nrows, pp.shape[0]), 1)
        p_oh = ((r_p // omod) % pmod == c_p).astype(f32)
        r_o = lax.broadcasted_iota(jnp.int32, (nrows, op.shape[0]), 0)
        c_o = lax.broadcasted_iota(jnp.int32, (nrows, op.shape[0]), 1)
        o_oh = (r_o % omod == c_o).astype(f32)
        return (lax.dot_general(s_oh, sp, dn2, preferred_element_type=f32)
                + lax.dot_general(p_oh, pp, dn2, preferred_element_type=f32)
                + lax.dot_general(o_oh, op, dn2, preferred_element_type=f32))

    b_row = bias[...]
    corner = combine(192, csp, cpp, cop, 8, 3) + b_row
    edge = combine(288, esp, epp, eop, 12, 2) + b_row
    zeros = jnp.zeros((32, 128), f32)
    table_out[0:192, :] = corner[:, :128]
    table_out[192:480, :] = edge[:, :128]
    table_out[480:512, :] = zeros
    table_out[512:704, :] = corner[:, 128:]
    table_out[704:992, :] = edge[:, 128:]
    table_out[992:1024, :] = zeros

    r_it = lax.broadcasted_iota(jnp.int32, (N // 64, 64), 0)
    t = r_it // 256
    s_id = slot[...]
    p_id = piece[...]
    o_id = orient[...]
    is_c = t < 8
    corner_idx = s_id * 24 + p_id * 3 + o_id
    edge_idx = 192 + s_id * 24 + p_id * 2 + o_id
    base = jnp.where(is_c, corner_idx, edge_idx).astype(f32)

    po = lax.broadcasted_iota(jnp.int32, (64, 128), 0)
    pc = lax.broadcasted_iota(jnp.int32, (64, 128), 1)
    perm = (po == 8 * (pc // 16) + (pc % 8)).astype(f32)
    c1 = lax.broadcasted_iota(jnp.int32, (1, 128), 1)
    halfoff = ((c1 // 8) % 2) * (TBL_ROWS // 2)
    expanded = lax.dot_general(base, perm, dn2, preferred_element_type=f32,
                               precision=lax.Precision.HIGHEST)
    idx_out[...] = (expanded + 0.5).astype(jnp.int32) + halfoff


def _gather_body(tbl_hbm, idx_hbm, out_hbm,
                 idx_v, rows_v, tbl_sh, g0, g1, g2, g3, o0, o1, o2, o3):
    gsem = [g0, g1, g2, g3]
    osem = [o0, o1, o2, o3]
    wid = lax.axis_index("s") * NC + lax.axis_index("c")
    base = wid * OUT_PER_W

    sid = lax.axis_index("s")

    @pl.when(sid < TBL_ROWS // CHUNK)
    def _stage():
        sl = pl.ds(sid * CHUNK, CHUNK)
        pltpu.sync_copy(tbl_hbm.at[sl], rows_v.at[0])
        pltpu.sync_copy(rows_v.at[0], tbl_sh.at[sl])

    pltpu.sync_copy(idx_hbm.at[wid], idx_v)
    plsc.subcore_barrier()

    def g_start(j, b):
        pltpu.async_copy(tbl_sh.at[idx_v.at[j]], rows_v.at[b], gsem[b])

    def g_wait(j, b):
        pltpu.make_async_copy(
            tbl_sh.at[idx_v.at[j]], rows_v.at[b], gsem[b]).wait()

    def o_start(j, b):
        dst = pl.ds(base + j * CHUNK, CHUNK)
        pltpu.async_copy(rows_v.at[b], out_hbm.at[dst], osem[b])

    def o_wait(j, b):
        dst = pl.ds(base + j * CHUNK, CHUNK)
        pltpu.make_async_copy(rows_v.at[b], out_hbm.at[dst], osem[b]).wait()

    for b in range(NBUF):
        g_start(b, b)

    def group(g, carry):
        for b in range(NBUF):
            j = g * NBUF + b
            g_wait(j, b)
            o_start(j, b)

            @pl.when(g < NGROUP - 1)
            def _refill():
                o_wait(j, b)
                g_start(j + NBUF, b)

        return carry

    lax.fori_loop(0, NGROUP, group, 0)
    for b in range(NBUF):
        o_wait((NGROUP - 1) * NBUF + b, b)


def kernel(slot_ids, piece_ids, orientations, corner_slot_emb,
           corner_piece_emb, corner_orient_emb, edge_slot_emb,
           edge_piece_emb, edge_orient_emb, proj_W, proj_b):
    f32 = jnp.float32

    def pad_cols(a, lo):
        return jnp.pad(a, ((0, 0), (lo, 128 - lo - a.shape[1])))

    cs = pad_cols(corner_slot_emb, 0)
    cp = pad_cols(corner_piece_emb, 42)
    co = pad_cols(corner_orient_emb, 84)
    es = pad_cols(edge_slot_emb, 0)
    ep = pad_cols(edge_piece_emb, 42)
    eo = pad_cols(edge_orient_emb, 84)

    slot_r = slot_ids.transpose(1, 0).reshape(N // 64, 64)
    piece_r = piece_ids.transpose(1, 0).reshape(N // 64, 64)
    orient_r = orientations.transpose(1, 0).reshape(N // 64, 64)

    table, idx = pl.pallas_call(
        _prep_body,
        out_shape=[
            jax.ShapeDtypeStruct((TBL_ROWS, 128), f32),
            jax.ShapeDtypeStruct((N // 64, 128), jnp.int32),
        ],
    )(cs, cp, co, es, ep, eo, proj_W, proj_b.reshape(1, D_OUT),
      slot_r, piece_r, orient_r)

    idx_3d = idx.reshape(NW, NCHUNK, CHUNK)

    mesh = plsc.VectorSubcoreMesh(core_axis_name="c", subcore_axis_name="s")
    gather = functools.partial(
        pl.kernel,
        mesh=mesh,
        out_type=jax.ShapeDtypeStruct((2 * N, 128), f32),
        scratch_types=[
            pltpu.VMEM((NCHUNK, CHUNK), jnp.int32),
            pltpu.VMEM((NBUF, CHUNK, 128), f32),
            pltpu.VMEM_SHARED((TBL_ROWS, 128), f32),
            pltpu.SemaphoreType.DMA,
            pltpu.SemaphoreType.DMA,
            pltpu.SemaphoreType.DMA,
            pltpu.SemaphoreType.DMA,
            pltpu.SemaphoreType.DMA,
            pltpu.SemaphoreType.DMA,
            pltpu.SemaphoreType.DMA,
            pltpu.SemaphoreType.DMA,
        ],
    )(_gather_body)

    out = gather(table, idx_3d)

    return (out.reshape(T, B // 8, 2, 8, 128)
            .transpose(1, 3, 0, 2, 4)
            .reshape(B, T, D_OUT))


# speedup vs baseline: 12.2173x; 1.0499x over previous
import functools

import jax
import jax.numpy as jnp
from jax import lax
from jax.experimental import pallas as pl
from jax.experimental.pallas import tpu as pltpu
from jax.experimental.pallas import tpu_sc as plsc

B = 16384
T = 20
N = B * T
D_OUT = 256
TBL_ROWS = 1024
NC = 2
NS = 16
NW = NC * NS
TOK_PER_W = N // NW
OUT_PER_W = 2 * TOK_PER_W
CHUNK = 128
TOK_PER_CHUNK = CHUNK // 2
NCHUNK = OUT_PER_W // CHUNK
NBUF = 4
NGROUP = NCHUNK // NBUF


def _prep_body(cs, cp, co, es, ep, eo, w, bias, slot, piece, orient,
               table_out, idx_out):
    dn = (((1,), (1,)), ((), ()))
    f32 = jnp.float32
    wv = w[...]
    w_s, w_p, w_o = wv[:, 0:42], wv[:, 42:84], wv[:, 84:128]
    csp = lax.dot_general(cs[...], w_s, dn, preferred_element_type=f32)
    cpp = lax.dot_general(cp[...], w_p, dn, preferred_element_type=f32)
    cop = lax.dot_general(co[...], w_o, dn, preferred_element_type=f32)
    esp = lax.dot_general(es[...], w_s, dn, preferred_element_type=f32)
    epp = lax.dot_general(ep[...], w_p, dn, preferred_element_type=f32)
    eop = lax.dot_general(eo[...], w_o, dn, preferred_element_type=f32)

    dn2 = (((1,), (0,)), ((), ()))

    def combine(nrows, sp, pp, op, pmod, omod):
        r_s = lax.broadcasted_iota(jnp.int32, (nrows, sp.shape[0]), 0)
        c_s = lax.broadcasted_iota(jnp.int32, (nrows, sp.shape[0]), 1)
        s_oh = (r_s // (pmod * omod) == c_s).astype(f32)
        r_p = lax.broadcasted_iota(jnp.int32, (nrows, pp.shape[0]), 0)
        c_p = lax.broadcasted_iota(jnp.int32, (nrows, pp.shape[0]), 1)
        p_oh = ((r_p // omod) % pmod == c_p).astype(f32)
        r_o = lax.broadcasted_iota(jnp.int32, (nrows, op.shape[0]), 0)
        c_o = lax.broadcasted_iota(jnp.int32, (nrows, op.shape[0]), 1)
        o_oh = (r_o % omod == c_o).astype(f32)
        return (lax.dot_general(s_oh, sp, dn2, preferred_element_type=f32)
                + lax.dot_general(p_oh, pp, dn2, preferred_element_type=f32)
                + lax.dot_general(o_oh, op, dn2, preferred_element_type=f32))

    b_row = bias[...]
    corner = combine(192, csp, cpp, cop, 8, 3) + b_row
    edge = combine(288, esp, epp, eop, 12, 2) + b_row
    zeros = jnp.zeros((32, 128), f32)
    table_out[0:192, :] = corner[:, :128]
    table_out[192:480, :] = edge[:, :128]
    table_out[480:512, :] = zeros
    table_out[512:704, :] = corner[:, 128:]
    table_out[704:992, :] = edge[:, 128:]
    table_out[992:1024, :] = zeros

    r_it = lax.broadcasted_iota(jnp.int32, (N // 64, 64), 0)
    t = r_it // 256
    s_id = slot[...]
    p_id = piece[...]
    o_id = orient[...]
    is_c = t < 8
    corner_idx = s_id * 24 + p_id * 3 + o_id
    edge_idx = 192 + s_id * 24 + p_id * 2 + o_id
    base = jnp.where(is_c, corner_idx, edge_idx).astype(f32)

    po = lax.broadcasted_iota(jnp.int32, (64, 128), 0)
    pc = lax.broadcasted_iota(jnp.int32, (64, 128), 1)
    perm = (po == 8 * (pc // 16) + (pc % 8)).astype(f32)
    c1 = lax.broadcasted_iota(jnp.int32, (1, 128), 1)
    halfoff = ((c1 // 8) % 2) * (TBL_ROWS // 2)
    expanded = lax.dot_general(base, perm, dn2, preferred_element_type=f32,
                               precision=lax.Precision.HIGHEST)
    idx_out[...] = (expanded + 0.5).astype(jnp.int32) + halfoff


def _gather_body(tbl_hbm, idx_hbm, out_hbm,
                 idx_v, rows_v, tbl_sh, g0, g1, g2, g3, o0, o1, o2, o3):
    gsem = [g0, g1, g2, g3]
    osem = [o0, o1, o2, o3]
    wid = lax.axis_index("s") * NC + lax.axis_index("c")
    base = wid * OUT_PER_W

    sid = lax.axis_index("s")

    @pl.when(sid < TBL_ROWS // CHUNK)
    def _stage():
        sl = pl.ds(sid * CHUNK, CHUNK)
        pltpu.sync_copy(tbl_hbm.at[sl], rows_v.at[0])
        pltpu.sync_copy(rows_v.at[0], tbl_sh.at[sl])

    pltpu.sync_copy(idx_hbm.at[wid], idx_v)
    plsc.subcore_barrier()

    def g_start(j, b):
        pltpu.async_copy(tbl_sh.at[idx_v.at[j]], rows_v.at[b], gsem[b])

    def g_wait(j, b):
        pltpu.make_async_copy(
            tbl_sh.at[idx_v.at[j]], rows_v.at[b], gsem[b]).wait()

    def o_start(j, b):
        dst = pl.ds(base + j * CHUNK, CHUNK)
        pltpu.async_copy(rows_v.at[b], out_hbm.at[dst], osem[b])

    def o_wait(j, b):
        dst = pl.ds(base + j * CHUNK, CHUNK)
        pltpu.make_async_copy(rows_v.at[b], out_hbm.at[dst], osem[b]).wait()

    for b in range(NBUF):
        g_start(b, b)

    def group(g, carry):
        for b in range(NBUF):
            j = g * NBUF + b
            g_wait(j, b)
            o_start(j, b)

            @pl.when(g < NGROUP - 1)
            def _refill():
                o_wait(j, b)
                g_start(j + NBUF, b)

        return carry

    lax.fori_loop(0, NGROUP, group, 0)
    for b in range(NBUF):
        o_wait((NGROUP - 1) * NBUF + b, b)


def kernel(slot_ids, piece_ids, orientations, corner_slot_emb,
           corner_piece_emb, corner_orient_emb, edge_slot_emb,
           edge_piece_emb, edge_orient_emb, proj_W, proj_b):
    f32 = jnp.float32

    cs, cp, co = corner_slot_emb, corner_piece_emb, corner_orient_emb
    es, ep, eo = edge_slot_emb, edge_piece_emb, edge_orient_emb

    slot_r = slot_ids.transpose(1, 0).reshape(N // 64, 64)
    piece_r = piece_ids.transpose(1, 0).reshape(N // 64, 64)
    orient_r = orientations.transpose(1, 0).reshape(N // 64, 64)

    table, idx = pl.pallas_call(
        _prep_body,
        out_shape=[
            jax.ShapeDtypeStruct((TBL_ROWS, 128), f32),
            jax.ShapeDtypeStruct((N // 64, 128), jnp.int32),
        ],
    )(cs, cp, co, es, ep, eo, proj_W, proj_b.reshape(1, D_OUT),
      slot_r, piece_r, orient_r)

    idx_3d = idx.reshape(NW, NCHUNK, CHUNK)

    mesh = plsc.VectorSubcoreMesh(core_axis_name="c", subcore_axis_name="s")
    gather = functools.partial(
        pl.kernel,
        mesh=mesh,
        out_type=jax.ShapeDtypeStruct((2 * N, 128), f32),
        scratch_types=[
            pltpu.VMEM((NCHUNK, CHUNK), jnp.int32),
            pltpu.VMEM((NBUF, CHUNK, 128), f32),
            pltpu.VMEM_SHARED((TBL_ROWS, 128), f32),
            pltpu.SemaphoreType.DMA,
            pltpu.SemaphoreType.DMA,
            pltpu.SemaphoreType.DMA,
            pltpu.SemaphoreType.DMA,
            pltpu.SemaphoreType.DMA,
            pltpu.SemaphoreType.DMA,
            pltpu.SemaphoreType.DMA,
            pltpu.SemaphoreType.DMA,
        ],
    )(_gather_body)

    out = gather(table, idx_3d)

    return (out.reshape(T, B // 8, 2, 8, 128)
            .transpose(1, 3, 0, 2, 4)
            .reshape(B, T, D_OUT))
